# baseline (speedup 1.0000x reference)
import jax
import jax.numpy as jnp
from jax.experimental import pallas as pl


def _concat_kernel(d_ref, e_ref, o_ref):
    tt = d_ref.shape[1]
    f = d_ref.shape[2]
    e = e_ref.shape[1]
    n = d_ref.shape[3]
    o_ref[0, :, :f, :] = d_ref[0]
    for tl in range(tt):
        col = e_ref[0, :, tl]
        o_ref[0, tl, f:, :] = jnp.broadcast_to(col[:, None], (e, n))


def kernel(data, emb):
    B, N, T, F = data.shape
    _, E = emb.shape
    W = F + E
    TT = 4
    dataT = jnp.transpose(data, (0, 2, 3, 1))
    emb3 = emb.T.reshape(E, T // TT, TT).swapaxes(0, 1)

    out = pl.pallas_call(
        _concat_kernel,
        grid=(B, T // TT),
        in_specs=[
            pl.BlockSpec((1, TT, F, N), lambda b, t: (b, t, 0, 0)),
            pl.BlockSpec((1, E, TT), lambda b, t: (t, 0, 0)),
        ],
        out_specs=pl.BlockSpec((1, TT, W, N), lambda b, t: (b, t, 0, 0)),
        out_shape=jax.ShapeDtypeStruct((B, T, W, N), jnp.float32),
    )(dataT, emb3)
    return jnp.transpose(out, (0, 3, 1, 2))


# speedup vs baseline: 1.0903x; 1.0903x over previous
import jax
import jax.numpy as jnp
from jax.experimental import pallas as pl
from jax.experimental.pallas import tpu as pltpu


def _concat_kernel(d_ref, e_ref, o_ref):
    tt = d_ref.shape[1]
    f = d_ref.shape[2]
    e = e_ref.shape[1]
    n = d_ref.shape[3]
    o_ref[0, :, :f, :] = d_ref[0]
    for tl in range(tt):
        col = e_ref[0, :, tl]
        o_ref[0, tl, f:, :] = jnp.broadcast_to(col[:, None], (e, n))


def kernel(data, emb):
    B, N, T, F = data.shape
    _, E = emb.shape
    W = F + E
    TT = 16
    dataT = jnp.transpose(data, (0, 2, 3, 1))
    emb3 = emb.T.reshape(E, T // TT, TT).swapaxes(0, 1)

    out = pl.pallas_call(
        _concat_kernel,
        grid=(B, T // TT),
        in_specs=[
            pl.BlockSpec((1, TT, F, N), lambda b, t: (b, t, 0, 0)),
            pl.BlockSpec((1, E, TT), lambda b, t: (t, 0, 0)),
        ],
        out_specs=pl.BlockSpec((1, TT, W, N), lambda b, t: (b, t, 0, 0)),
        out_shape=jax.ShapeDtypeStruct((B, T, W, N), jnp.float32),
        compiler_params=pltpu.CompilerParams(
            dimension_semantics=("parallel", "parallel")
        ),
    )(dataT, emb3)
    return jnp.transpose(out, (0, 3, 1, 2))
